# Initial kernel scaffold; baseline (speedup 1.0000x reference)
import jax
import jax.numpy as jnp
from jax.experimental import pallas as pl


def kernel(means2D, cov2d, color, opacity, depths):
    raise NotImplementedError("write your pallas kernel here")


# speedup vs baseline: 9.6678x; 9.6678x over previous
import jax
import jax.numpy as jnp
from jax.experimental import pallas as pl

H = 128
W = 128
TILE = 16
N = 1024
K = 256
P = TILE * W
BKGD = 1.0


def _raster_kernel(attrs_ref, covu_ref, color_ref, alpha_ref, radii_ref):
    pid = pl.program_id(0)
    h0 = (pid * TILE).astype(jnp.float32)

    @pl.when(pid == 0)
    def _():
        ca = covu_ref[0:1, :]
        cb = covu_ref[1:2, :]
        cd = covu_ref[2:3, :]
        det = ca * cd - cb * cb
        mid = 0.5 * (ca + cd)
        root = jnp.sqrt(jnp.maximum(mid * mid - det, 0.1))
        radii_ref[0:1, :] = jnp.ceil(3.0 * jnp.sqrt(mid + root))

    mx = attrs_ref[0:1, :]
    my = attrs_ref[1:2, :]
    ca = attrs_ref[2:3, :]
    cb = attrs_ref[3:4, :]
    cd = attrs_ref[4:5, :]
    op = attrs_ref[5:6, :]
    colr = attrs_ref[6:7, :]
    colg = attrs_ref[7:8, :]
    colb = attrs_ref[8:9, :]

    det = ca * cd - cb * cb
    i00 = cd / det
    i11 = ca / det
    i01 = -cb / det
    mid = 0.5 * (ca + cd)
    root = jnp.sqrt(jnp.maximum(mid * mid - det, 0.1))
    rad = jnp.ceil(3.0 * jnp.sqrt(mid + root))
    rminx = jnp.clip(mx - rad, 0.0, W - 1.0)
    rmaxx = jnp.clip(mx + rad, 0.0, W - 1.0)
    rminy = jnp.clip(my - rad, 0.0, H - 1.0)
    rmaxy = jnp.clip(my + rad, 0.0, H - 1.0)
    masky = (jnp.minimum(rmaxy, h0 + (TILE - 1.0)) >
             jnp.maximum(rminy, h0)).astype(jnp.float32)

    ii = jax.lax.broadcasted_iota(jnp.int32, (P, 1), 0)
    pxi = ii % W
    pxf = pxi.astype(jnp.float32)
    pyf = h0 + (ii // W).astype(jnp.float32)
    woxf = ((pxi // TILE) * TILE).astype(jnp.float32)

    triu = (jax.lax.broadcasted_iota(jnp.int32, (K, K), 0) <
            jax.lax.broadcasted_iota(jnp.int32, (K, K), 1)).astype(jnp.float32)

    t_carry = jnp.ones((P, 1), jnp.float32)
    acc_a = jnp.zeros((P, 1), jnp.float32)
    acc_r = jnp.zeros((P, 1), jnp.float32)
    acc_g = jnp.zeros((P, 1), jnp.float32)
    acc_b = jnp.zeros((P, 1), jnp.float32)

    for c in range(N // K):
        sl = slice(c * K, (c + 1) * K)
        mxk = mx[:, sl]
        myk = my[:, sl]
        dx = pxf - mxk
        dy = pyf - myk
        quad = (dx * dx * i00[:, sl] + dy * dy * i11[:, sl]
                + 2.0 * (dx * dy) * i01[:, sl])
        gw = jnp.exp(-0.5 * quad)
        maskx = (jnp.minimum(rmaxx[:, sl], woxf + (TILE - 1.0)) >
                 jnp.maximum(rminx[:, sl], woxf)).astype(jnp.float32)
        alpha = jnp.minimum(gw * op[:, sl], 0.99) * maskx * masky[:, sl]
        lg = jnp.log1p(-alpha)
        sexc = jax.lax.dot_general(
            lg, triu, (((1,), (0,)), ((), ())),
            precision=jax.lax.Precision.HIGHEST,
            preferred_element_type=jnp.float32)
        t_in = t_carry * jnp.exp(sexc)
        wgt = t_in * alpha
        acc_a = acc_a + jnp.sum(wgt, axis=1, keepdims=True)
        acc_r = acc_r + jnp.sum(wgt * colr[:, sl], axis=1, keepdims=True)
        acc_g = acc_g + jnp.sum(wgt * colg[:, sl], axis=1, keepdims=True)
        acc_b = acc_b + jnp.sum(wgt * colb[:, sl], axis=1, keepdims=True)
        t_carry = t_carry * jnp.exp(jnp.sum(lg, axis=1, keepdims=True))

    resid = (1.0 - acc_a) * BKGD
    color_ref[0, :, :] = jnp.reshape(acc_r + resid, (TILE, W))
    color_ref[1, :, :] = jnp.reshape(acc_g + resid, (TILE, W))
    color_ref[2, :, :] = jnp.reshape(acc_b + resid, (TILE, W))
    alpha_ref[0, :, :] = jnp.reshape(acc_a, (TILE, W))


def _rasterize(attrs, covu):
    return pl.pallas_call(
        _raster_kernel,
        grid=(H // TILE,),
        in_specs=[
            pl.BlockSpec((16, N), lambda i: (0, 0)),
            pl.BlockSpec((8, N), lambda i: (0, 0)),
        ],
        out_specs=[
            pl.BlockSpec((3, TILE, W), lambda i: (0, i, 0)),
            pl.BlockSpec((1, TILE, W), lambda i: (0, i, 0)),
            pl.BlockSpec((1, N), lambda i: (0, 0)),
        ],
        out_shape=[
            jax.ShapeDtypeStruct((3, H, W), jnp.float32),
            jax.ShapeDtypeStruct((1, H, W), jnp.float32),
            jax.ShapeDtypeStruct((1, N), jnp.float32),
        ],
    )(attrs, covu)


@jax.jit
def kernel(means2D, cov2d, color, opacity, depths):
    order = jnp.argsort(depths)
    attrs = jnp.stack([
        means2D[:, 0], means2D[:, 1],
        cov2d[:, 0, 0], cov2d[:, 0, 1], cov2d[:, 1, 1],
        opacity[:, 0],
        color[:, 0], color[:, 1], color[:, 2],
    ], axis=0)[:, order]
    attrs = jnp.concatenate(
        [attrs, jnp.zeros((16 - attrs.shape[0], N), jnp.float32)], axis=0)
    covu = jnp.stack([cov2d[:, 0, 0], cov2d[:, 0, 1], cov2d[:, 1, 1]], axis=0)
    covu = jnp.concatenate(
        [covu, jnp.zeros((8 - covu.shape[0], N), jnp.float32)], axis=0)
    col, alp, rad = _rasterize(attrs, covu)
    return (jnp.transpose(col, (1, 2, 0)),
            jnp.transpose(alp, (1, 2, 0)),
            rad[0])
